# baseline (speedup 1.0000x reference)
import functools

import jax
import jax.numpy as jnp
from jax import lax
from jax.experimental import pallas as pl
from jax.experimental.pallas import tpu as pltpu
from jax.experimental.pallas import tpu_sc as plsc

_NSC = 2
_NTILE = 16
_LANES = 16
_CH = 80


def _sc_segment_sum(hcat, src, dst, z128, ones128):
  n2, dh = hcat.shape
  n = n2 // 2
  np_ = z128.shape[0]
  e = src.shape[0]
  epw = e // _NTILE
  iters = epw // _CH
  assert epw % _CH == 0 and np_ % (8 * _NTILE) == 0
  rpt = np_ // _NTILE
  full_chunks, tail = divmod(rpt, _CH)
  assert tail % 8 == 0

  mesh = plsc.VectorSubcoreMesh(
      core_axis_name="c", subcore_axis_name="s",
      num_cores=_NSC, num_subcores=_NTILE)

  @functools.partial(
      pl.kernel,
      out_type=[
          jax.ShapeDtypeStruct((2 * np_, dh), jnp.float32),
          jax.ShapeDtypeStruct((np_, dh), jnp.float32),
      ],
      mesh=mesh,
      scratch_types=[
          pltpu.VMEM((_CH,), jnp.int32),
          pltpu.VMEM((_CH,), jnp.int32),
          pltpu.VMEM((_CH, dh), jnp.float32),
          pltpu.VMEM((_CH, dh), jnp.float32),
          pltpu.VMEM_SHARED((np_, dh), jnp.float32),
          pltpu.SemaphoreType.DMA,
      ])
  def kern(hcat_hbm, src_hbm, dst_hbm, z128_hbm, ones_hbm,
           msg_hbm, cnt_hbm, srcb, dstb, rows, ones, acc, sem):
    c = lax.axis_index("c")
    s = lax.axis_index("s")
    r0 = s * rpt

    def zero_acc():
      for q in range(full_chunks):
        pltpu.sync_copy(z128_hbm.at[pl.ds(r0 + q * _CH, _CH)],
                        acc.at[pl.ds(r0 + q * _CH, _CH)])
      if tail:
        t0 = r0 + full_chunks * _CH
        pltpu.sync_copy(z128_hbm.at[pl.ds(t0, tail)], acc.at[pl.ds(t0, tail)])

    def write_acc(out_hbm, base):
      for q in range(full_chunks):
        pltpu.sync_copy(acc.at[pl.ds(r0 + q * _CH, _CH)],
                        out_hbm.at[pl.ds(base + r0 + q * _CH, _CH)])
      if tail:
        t0 = r0 + full_chunks * _CH
        pltpu.sync_copy(acc.at[pl.ds(t0, tail)],
                        out_hbm.at[pl.ds(base + t0, tail)])

    zero_acc()
    plsc.subcore_barrier()

    def body(k, carry):
      off = s * epw + k * _CH
      pltpu.sync_copy(src_hbm.at[pl.ds(off, _CH)], srcb)
      pltpu.sync_copy(dst_hbm.at[pl.ds(off, _CH)], dstb)
      for i in range(_CH // _LANES):
        srcb[pl.ds(i * _LANES, _LANES)] = (
            srcb[pl.ds(i * _LANES, _LANES)] + c * n)
      pltpu.async_copy(hcat_hbm.at[srcb], rows, sem).wait()
      pltpu.sync_copy(rows, acc.at[dstb], add=True)
      return carry

    lax.fori_loop(0, iters, body, 0)
    plsc.subcore_barrier()
    write_acc(msg_hbm, c * np_)
    plsc.subcore_barrier()

    @pl.when(c == 0)
    def _():
      zero_acc()
      pltpu.sync_copy(ones_hbm, ones)
      plsc.subcore_barrier()

      def cbody(k, carry):
        off = s * epw + k * _CH
        pltpu.sync_copy(dst_hbm.at[pl.ds(off, _CH)], dstb)
        pltpu.sync_copy(ones, acc.at[dstb], add=True)
        return carry

      lax.fori_loop(0, iters, cbody, 0)
      plsc.subcore_barrier()
      write_acc(cnt_hbm, 0)

  return kern(hcat, src, dst, z128, ones128)


def _tc_scores(msg0, msg1, h, cnt, Ws):
  n, d = h.shape
  c = Ws.shape[0]
  rows = 1000
  nb = n // rows
  assert n % rows == 0

  def body(m0, m1, hh, cc, ws, hag, idx):
    cnt1 = cc[:, 0:1] + 1.0
    ha = jnp.concatenate([m0[...], m1[...]], axis=1) + hh[...]
    ha = ha / cnt1
    hag[...] = ha
    s = lax.dot_general(ha, ws[...], (((1,), (1,)), ((), ())),
                        preferred_element_type=jnp.float32)
    idx[...] = jnp.argmax(s, axis=1).astype(jnp.int32).reshape(1, 1, rows)

  dh = d // 2
  return pl.pallas_call(
      body,
      grid=(nb,),
      in_specs=[
          pl.BlockSpec((rows, dh), lambda i: (i, 0)),
          pl.BlockSpec((rows, dh), lambda i: (i, 0)),
          pl.BlockSpec((rows, d), lambda i: (i, 0)),
          pl.BlockSpec((rows, d // 2), lambda i: (i, 0)),
          pl.BlockSpec((c, d), lambda i: (0, 0)),
      ],
      out_specs=[
          pl.BlockSpec((rows, d), lambda i: (i, 0)),
          pl.BlockSpec((1, 1, rows), lambda i: (i, 0, 0)),
      ],
      out_shape=[
          jax.ShapeDtypeStruct((n, d), jnp.float32),
          jax.ShapeDtypeStruct((nb, 1, rows), jnp.int32),
      ],
  )(msg0, msg1, h, cnt, Ws)


def _tc_task_heads(idx, h_agg, Wt):
  n, d = h_agg.shape
  nc = Wt.shape[1]
  ha3 = h_agg.reshape(n, 1, d)

  def body(idx_ref, ha_ref, wt_ref, out_ref):
    out_ref[...] = lax.dot_general(
        ha_ref[0], wt_ref[0], (((1,), (1,)), ((), ())),
        preferred_element_type=jnp.float32).reshape(1, 1, nc)

  grid_spec = pltpu.PrefetchScalarGridSpec(
      num_scalar_prefetch=1,
      grid=(n,),
      in_specs=[
          pl.BlockSpec((1, 1, d), lambda i, idx_ref: (i, 0, 0)),
          pl.BlockSpec((1, nc, d), lambda i, idx_ref: (idx_ref[i], 0, 0)),
      ],
      out_specs=pl.BlockSpec((1, 1, nc), lambda i, idx_ref: (i, 0, 0)),
  )
  out3 = pl.pallas_call(
      body,
      grid_spec=grid_spec,
      out_shape=jax.ShapeDtypeStruct((n, 1, nc), jnp.float32),
  )(idx, ha3, Wt)
  return out3.reshape(n, nc)


def kernel(h, edge_index, Ws, Wt):
  n, d = h.shape
  dh = d // 2
  src = edge_index[0]
  dst = edge_index[1]
  hcat = jnp.concatenate([h[:, :dh], h[:, dh:]], axis=0)
  np_ = ((n + 8 * _NTILE - 1) // (8 * _NTILE)) * (8 * _NTILE)
  z128 = jnp.zeros((np_, dh), jnp.float32)
  ones128 = jnp.ones((_CH, dh), jnp.float32)
  msg, cnt = _sc_segment_sum(hcat, src, dst, z128, ones128)
  h_agg, idx3 = _tc_scores(msg[:n], msg[np_:np_ + n], h, cnt[:n], Ws)
  idx = idx3.reshape(n)
  return _tc_task_heads(idx, h_agg, Wt)


# speedup vs baseline: 5.7036x; 5.7036x over previous
import functools

import jax
import jax.numpy as jnp
from jax import lax
from jax.experimental import pallas as pl
from jax.experimental.pallas import tpu as pltpu
from jax.experimental.pallas import tpu_sc as plsc

_NSC = 2
_NTILE = 16
_LANES = 16
_CH = 80


def _sc_segment_sum(hcat, src, dst, z128, ones128):
  n2, dh = hcat.shape
  n = n2 // 2
  np_ = z128.shape[0]
  e = src.shape[0]
  epw = e // _NTILE
  iters = epw // _CH
  assert epw % _CH == 0 and np_ % (8 * _NTILE) == 0
  rpt = np_ // _NTILE
  full_chunks, tail = divmod(rpt, _CH)
  assert tail % 8 == 0

  mesh = plsc.VectorSubcoreMesh(
      core_axis_name="c", subcore_axis_name="s",
      num_cores=_NSC, num_subcores=_NTILE)

  @functools.partial(
      pl.kernel,
      out_type=[
          jax.ShapeDtypeStruct((2 * np_, dh), jnp.float32),
          jax.ShapeDtypeStruct((np_, dh), jnp.float32),
      ],
      mesh=mesh,
      scratch_types=[
          pltpu.VMEM((_CH,), jnp.int32),
          pltpu.VMEM((_CH,), jnp.int32),
          pltpu.VMEM((_CH, dh), jnp.float32),
          pltpu.VMEM((_CH, dh), jnp.float32),
          pltpu.VMEM_SHARED((np_, dh), jnp.float32),
          pltpu.SemaphoreType.DMA,
      ])
  def kern(hcat_hbm, src_hbm, dst_hbm, z128_hbm, ones_hbm,
           msg_hbm, cnt_hbm, srcb, dstb, rows, ones, acc, sem):
    c = lax.axis_index("c")
    s = lax.axis_index("s")
    r0 = s * rpt

    def zero_acc():
      for q in range(full_chunks):
        pltpu.sync_copy(z128_hbm.at[pl.ds(r0 + q * _CH, _CH)],
                        acc.at[pl.ds(r0 + q * _CH, _CH)])
      if tail:
        t0 = r0 + full_chunks * _CH
        pltpu.sync_copy(z128_hbm.at[pl.ds(t0, tail)], acc.at[pl.ds(t0, tail)])

    def write_acc(out_hbm, base):
      for q in range(full_chunks):
        pltpu.sync_copy(acc.at[pl.ds(r0 + q * _CH, _CH)],
                        out_hbm.at[pl.ds(base + r0 + q * _CH, _CH)])
      if tail:
        t0 = r0 + full_chunks * _CH
        pltpu.sync_copy(acc.at[pl.ds(t0, tail)],
                        out_hbm.at[pl.ds(base + t0, tail)])

    zero_acc()
    plsc.subcore_barrier()

    def body(k, carry):
      off = s * epw + k * _CH
      pltpu.sync_copy(src_hbm.at[pl.ds(off, _CH)], srcb)
      pltpu.sync_copy(dst_hbm.at[pl.ds(off, _CH)], dstb)
      for i in range(_CH // _LANES):
        srcb[pl.ds(i * _LANES, _LANES)] = (
            srcb[pl.ds(i * _LANES, _LANES)] + c * n)
      pltpu.async_copy(hcat_hbm.at[srcb], rows, sem).wait()
      pltpu.sync_copy(rows, acc.at[dstb], add=True)
      return carry

    lax.fori_loop(0, iters, body, 0)
    plsc.subcore_barrier()
    write_acc(msg_hbm, c * np_)
    plsc.subcore_barrier()

    @pl.when(c == 0)
    def _():
      zero_acc()
      pltpu.sync_copy(ones_hbm, ones)
      plsc.subcore_barrier()

      def cbody(k, carry):
        off = s * epw + k * _CH
        pltpu.sync_copy(dst_hbm.at[pl.ds(off, _CH)], dstb)
        pltpu.sync_copy(ones, acc.at[dstb], add=True)
        return carry

      lax.fori_loop(0, iters, cbody, 0)
      plsc.subcore_barrier()
      write_acc(cnt_hbm, 0)

  return kern(hcat, src, dst, z128, ones128)


def _tc_scores(msg0, msg1, h, cnt, Ws):
  n, d = h.shape
  c = Ws.shape[0]
  rows = 1000
  nb = n // rows
  assert n % rows == 0

  def body(m0, m1, hh, cc, ws, hag, idx):
    cnt1 = cc[:, 0:1] + 1.0
    ha = jnp.concatenate([m0[...], m1[...]], axis=1) + hh[...]
    ha = ha / cnt1
    hag[...] = ha
    s = lax.dot_general(ha, ws[...], (((1,), (1,)), ((), ())),
                        preferred_element_type=jnp.float32)
    idx[...] = jnp.argmax(s, axis=1).astype(jnp.int32).reshape(1, 1, rows)

  dh = d // 2
  return pl.pallas_call(
      body,
      grid=(nb,),
      in_specs=[
          pl.BlockSpec((rows, dh), lambda i: (i, 0)),
          pl.BlockSpec((rows, dh), lambda i: (i, 0)),
          pl.BlockSpec((rows, d), lambda i: (i, 0)),
          pl.BlockSpec((rows, d // 2), lambda i: (i, 0)),
          pl.BlockSpec((c, d), lambda i: (0, 0)),
      ],
      out_specs=[
          pl.BlockSpec((rows, d), lambda i: (i, 0)),
          pl.BlockSpec((1, 1, rows), lambda i: (i, 0, 0)),
      ],
      out_shape=[
          jax.ShapeDtypeStruct((n, d), jnp.float32),
          jax.ShapeDtypeStruct((nb, 1, rows), jnp.int32),
      ],
  )(msg0, msg1, h, cnt, Ws)


def _tc_task_heads(idx, h_agg, Wt):
  n, d = h_agg.shape
  nc = Wt.shape[1]
  G = 16
  assert n % G == 0
  ha3 = h_agg.reshape(n, 1, d)

  def body(idx_ref, ha_ref, *rest):
    wt_refs = rest[:G]
    out_ref = rest[G]
    for k in range(G):
      out_ref[k] = lax.dot_general(
          ha_ref[k], wt_refs[k][0], (((1,), (1,)), ((), ())),
          preferred_element_type=jnp.float32)

  def mk_wt_spec(k):
    return pl.BlockSpec((1, nc, d),
                        lambda i, idx_ref, k=k: (idx_ref[G * i + k], 0, 0))

  grid_spec = pltpu.PrefetchScalarGridSpec(
      num_scalar_prefetch=1,
      grid=(n // G,),
      in_specs=[pl.BlockSpec((G, 1, d), lambda i, idx_ref: (i, 0, 0))]
      + [mk_wt_spec(k) for k in range(G)],
      out_specs=pl.BlockSpec((G, 1, nc), lambda i, idx_ref: (i, 0, 0)),
  )
  out3 = pl.pallas_call(
      body,
      grid_spec=grid_spec,
      out_shape=jax.ShapeDtypeStruct((n, 1, nc), jnp.float32),
  )(idx, ha3, *([Wt] * G))
  return out3.reshape(n, nc)


def kernel(h, edge_index, Ws, Wt):
  n, d = h.shape
  dh = d // 2
  src = edge_index[0]
  dst = edge_index[1]
  hcat = jnp.concatenate([h[:, :dh], h[:, dh:]], axis=0)
  np_ = ((n + 8 * _NTILE - 1) // (8 * _NTILE)) * (8 * _NTILE)
  z128 = jnp.zeros((np_, dh), jnp.float32)
  ones128 = jnp.ones((_CH, dh), jnp.float32)
  msg, cnt = _sc_segment_sum(hcat, src, dst, z128, ones128)
  h_agg, idx3 = _tc_scores(msg[:n], msg[np_:np_ + n], h, cnt[:n], Ws)
  idx = idx3.reshape(n)
  return _tc_task_heads(idx, h_agg, Wt)


# speedup vs baseline: 6.4112x; 1.1241x over previous
import functools

import jax
import jax.numpy as jnp
from jax import lax
from jax.experimental import pallas as pl
from jax.experimental.pallas import tpu as pltpu
from jax.experimental.pallas import tpu_sc as plsc

_NSC = 2
_NTILE = 16
_LANES = 16
_CH = 80


def _sc_segment_sum(hcat, src, dst, z128, ones128):
  n2, dh = hcat.shape
  n = n2 // 2
  np_ = z128.shape[0]
  e = src.shape[0]
  epw = e // _NTILE
  iters = epw // _CH
  assert epw % _CH == 0 and np_ % (8 * _NTILE) == 0
  rpt = np_ // _NTILE
  full_chunks, tail = divmod(rpt, _CH)
  assert tail % 8 == 0

  mesh = plsc.VectorSubcoreMesh(
      core_axis_name="c", subcore_axis_name="s",
      num_cores=_NSC, num_subcores=_NTILE)

  @functools.partial(
      pl.kernel,
      out_type=[
          jax.ShapeDtypeStruct((2 * np_, dh), jnp.float32),
          jax.ShapeDtypeStruct((2 * np_, dh), jnp.float32),
      ],
      mesh=mesh,
      scratch_types=[
          pltpu.VMEM((2, _CH), jnp.int32),
          pltpu.VMEM((2, _CH), jnp.int32),
          pltpu.VMEM((2, _CH, dh), jnp.float32),
          pltpu.VMEM((_CH, dh), jnp.float32),
          pltpu.VMEM((1, 40), jnp.int32),
          pltpu.VMEM_SHARED((np_, dh), jnp.float32),
          pltpu.SemaphoreType.DMA,
          pltpu.SemaphoreType.DMA,
      ])
  def kern(hcat_hbm, src_hbm, dst_hbm, z128_hbm, ones_hbm,
           msg_hbm, cnt_hbm, srcb, dstb, rows, ones, dstc, acc, sem0, sem1):
    c = lax.axis_index("c")
    s = lax.axis_index("s")
    r0 = s * rpt

    def zero_acc():
      for q in range(full_chunks):
        pltpu.sync_copy(z128_hbm.at[pl.ds(r0 + q * _CH, _CH)],
                        acc.at[pl.ds(r0 + q * _CH, _CH)])
      if tail:
        t0 = r0 + full_chunks * _CH
        pltpu.sync_copy(z128_hbm.at[pl.ds(t0, tail)], acc.at[pl.ds(t0, tail)])

    def write_acc(out_hbm, base):
      for q in range(full_chunks):
        pltpu.sync_copy(acc.at[pl.ds(r0 + q * _CH, _CH)],
                        out_hbm.at[pl.ds(base + r0 + q * _CH, _CH)])
      if tail:
        t0 = r0 + full_chunks * _CH
        pltpu.sync_copy(acc.at[pl.ds(t0, tail)],
                        out_hbm.at[pl.ds(base + t0, tail)])

    zero_acc()
    plsc.subcore_barrier()

    sems = (sem0, sem1)

    def load_start(k, b):
      off = s * epw + k * _CH
      pltpu.sync_copy(src_hbm.at[pl.ds(off, _CH)], srcb.at[b])
      pltpu.sync_copy(dst_hbm.at[pl.ds(off, _CH)], dstb.at[b])
      for i in range(_CH // _LANES):
        srcb[b, pl.ds(i * _LANES, _LANES)] = (
            srcb[b, pl.ds(i * _LANES, _LANES)] + c * n)
      return pltpu.async_copy(hcat_hbm.at[srcb.at[b]], rows.at[b], sems[b])

    def finish(k, b, cp):
      cp.wait()
      pltpu.sync_copy(rows.at[b], acc.at[dstb.at[b]], add=True)

    assert iters % 2 == 1 and iters >= 3
    load_start(0, 0).wait()
    pltpu.sync_copy(rows.at[0], acc.at[dstb.at[0]], add=True)

    def body(m, carry):
      k0 = 1 + 2 * m
      cp1 = load_start(k0, 1)
      cp0 = load_start(k0 + 1, 0)
      finish(k0, 1, cp1)
      finish(k0 + 1, 0, cp0)
      return carry

    lax.fori_loop(0, (iters - 1) // 2, body, 0)
    plsc.subcore_barrier()
    write_acc(msg_hbm, c * np_)
    plsc.subcore_barrier()

    zero_acc()
    pltpu.sync_copy(ones_hbm, ones)
    plsc.subcore_barrier()

    epw2 = e // (2 * _NTILE)
    ch2 = 40
    citer = epw2 // ch2
    assert epw2 % ch2 == 0

    def cbody(k, carry):
      off = c * (e // 2) + s * epw2 + k * ch2
      pltpu.sync_copy(dst_hbm.at[pl.ds(off, ch2)], dstc.at[0])
      pltpu.sync_copy(ones.at[pl.ds(0, ch2)], acc.at[dstc.at[0]], add=True)
      return carry

    lax.fori_loop(0, citer, cbody, 0)
    plsc.subcore_barrier()
    write_acc(cnt_hbm, c * np_)

  return kern(hcat, src, dst, z128, ones128)


def _tc_scores(msg0, msg1, h, cnt0, cnt1_, Ws):
  n, d = h.shape
  c = Ws.shape[0]
  rows = 1000
  nb = n // rows
  assert n % rows == 0

  def body(m0, m1, hh, cc0, cc1, ws, hag, idx):
    cnt1 = cc0[:, 0:1] + cc1[:, 0:1] + 1.0
    ha = jnp.concatenate([m0[...], m1[...]], axis=1) + hh[...]
    ha = ha / cnt1
    hag[...] = ha
    s = lax.dot_general(ha, ws[...], (((1,), (1,)), ((), ())),
                        preferred_element_type=jnp.float32)
    idx[...] = jnp.argmax(s, axis=1).astype(jnp.int32).reshape(1, 1, rows)

  dh = d // 2
  return pl.pallas_call(
      body,
      grid=(nb,),
      in_specs=[
          pl.BlockSpec((rows, dh), lambda i: (i, 0)),
          pl.BlockSpec((rows, dh), lambda i: (i, 0)),
          pl.BlockSpec((rows, d), lambda i: (i, 0)),
          pl.BlockSpec((rows, d // 2), lambda i: (i, 0)),
          pl.BlockSpec((rows, d // 2), lambda i: (i, 0)),
          pl.BlockSpec((c, d), lambda i: (0, 0)),
      ],
      out_specs=[
          pl.BlockSpec((rows, d), lambda i: (i, 0)),
          pl.BlockSpec((1, 1, rows), lambda i: (i, 0, 0)),
      ],
      out_shape=[
          jax.ShapeDtypeStruct((n, d), jnp.float32),
          jax.ShapeDtypeStruct((nb, 1, rows), jnp.int32),
      ],
  )(msg0, msg1, h, cnt0, cnt1_, Ws)


def _tc_task_heads(idx, h_agg, Wt):
  n, d = h_agg.shape
  nc = Wt.shape[1]
  G = 16
  assert n % G == 0
  ha3 = h_agg.reshape(n, 1, d)

  def body(idx_ref, ha_ref, *rest):
    wt_refs = rest[:G]
    out_ref = rest[G]
    for k in range(G):
      out_ref[k] = lax.dot_general(
          ha_ref[k], wt_refs[k][0], (((1,), (1,)), ((), ())),
          preferred_element_type=jnp.float32)

  def mk_wt_spec(k):
    return pl.BlockSpec((1, nc, d),
                        lambda i, idx_ref, k=k: (idx_ref[G * i + k], 0, 0))

  grid_spec = pltpu.PrefetchScalarGridSpec(
      num_scalar_prefetch=1,
      grid=(n // G,),
      in_specs=[pl.BlockSpec((G, 1, d), lambda i, idx_ref: (i, 0, 0))]
      + [mk_wt_spec(k) for k in range(G)],
      out_specs=pl.BlockSpec((G, 1, nc), lambda i, idx_ref: (i, 0, 0)),
  )
  out3 = pl.pallas_call(
      body,
      grid_spec=grid_spec,
      out_shape=jax.ShapeDtypeStruct((n, 1, nc), jnp.float32),
  )(idx, ha3, *([Wt] * G))
  return out3.reshape(n, nc)


def kernel(h, edge_index, Ws, Wt):
  n, d = h.shape
  dh = d // 2
  src = edge_index[0]
  dst = edge_index[1]
  hcat = jnp.concatenate([h[:, :dh], h[:, dh:]], axis=0)
  np_ = ((n + 8 * _NTILE - 1) // (8 * _NTILE)) * (8 * _NTILE)
  z128 = jnp.zeros((np_, dh), jnp.float32)
  ones128 = jnp.ones((_CH, dh), jnp.float32)
  msg, cnt = _sc_segment_sum(hcat, src, dst, z128, ones128)
  h_agg, idx3 = _tc_scores(msg[:n], msg[np_:np_ + n], h,
                           cnt[:n], cnt[np_:np_ + n], Ws)
  idx = idx3.reshape(n)
  return _tc_task_heads(idx, h_agg, Wt)
